# baseline (speedup 1.0000x reference)
import functools

import jax
import jax.numpy as jnp
from jax import lax
from jax.experimental import pallas as pl
from jax.experimental.pallas import tpu as pltpu
from jax.experimental.pallas import tpu_sc as plsc

N = 10000
E = 320000
F_IN = 128
H = 64
C = 7

NB = 10
BN = N // NB
BE = 2000
NWORK = 32
EW = E // NWORK
B = 80
EPS = 1e-30


def _lrelu_pos(t):
    return jnp.where(t >= 0.0, t, 0.2 * t)


def _lrelu_neg(t):
    return jnp.where(t <= 0.0, t, 0.2 * t)


def _prep_body(c2d, w11, w12, w21d, w22d, par1, par2):
    cabs = jnp.max(jnp.abs(c2d[...]))
    up1 = jnp.dot(_lrelu_pos(w11[...]), w12[...], preferred_element_type=jnp.float32)
    un1 = jnp.dot(_lrelu_neg(w11[...]), w12[...], preferred_element_type=jnp.float32)
    m1 = cabs * jnp.maximum(jnp.abs(up1), jnp.abs(un1))
    par1[...] = jnp.concatenate([up1, un1, m1, jnp.zeros((5, H), jnp.float32)], axis=0)
    up2 = jnp.dot(_lrelu_pos(w21d[...]), w22d[...], preferred_element_type=jnp.float32)
    un2 = jnp.dot(_lrelu_neg(w21d[...]), w22d[...], preferred_element_type=jnp.float32)
    m2 = cabs * jnp.maximum(jnp.abs(up2), jnp.abs(un2))
    par2[...] = jnp.concatenate([up2, un2, m2, jnp.zeros((5, 16), jnp.float32)], axis=0)


def _prep(c, w11, w12, w21d, w22d):
    c2d = c.reshape(2500, 128)
    return pl.pallas_call(
        _prep_body,
        out_shape=(
            jax.ShapeDtypeStruct((8, H), jnp.float32),
            jax.ShapeDtypeStruct((8, 16), jnp.float32),
        ),
    )(c2d, w11, w12, w21d, w22d)


def _wexp_body(c_ref, par_ref, o_ref):
    cb = c_ref[...]
    up = par_ref[0:1, :]
    un = par_ref[1:2, :]
    m = par_ref[2:3, :]
    o_ref[...] = jnp.exp(
        jnp.maximum(cb, 0.0) * up + jnp.minimum(cb, 0.0) * un - m
    )


def _wexp(c1, par, w):
    return pl.pallas_call(
        _wexp_body,
        grid=(E // BE,),
        in_specs=[
            pl.BlockSpec((BE, 1), lambda i: (i, 0)),
            pl.BlockSpec((8, w), lambda i: (0, 0)),
        ],
        out_specs=pl.BlockSpec((BE, w), lambda i: (i, 0)),
        out_shape=jax.ShapeDtypeStruct((E, w), jnp.float32),
    )(c1, par)


def _h1_body(x_ref, w_ref, b_ref, o_ref):
    o_ref[...] = (
        jnp.dot(x_ref[...], w_ref[...], preferred_element_type=jnp.float32)
        + b_ref[...]
    )


def _h1_matmul(x, w, b):
    return pl.pallas_call(
        _h1_body,
        grid=(NB,),
        in_specs=[
            pl.BlockSpec((BN, F_IN), lambda i: (i, 0)),
            pl.BlockSpec((F_IN, H), lambda i: (0, 0)),
            pl.BlockSpec((1, H), lambda i: (0, 0)),
        ],
        out_specs=pl.BlockSpec((BN, H), lambda i: (i, 0)),
        out_shape=jax.ShapeDtypeStruct((N, H), jnp.float32),
    )(x, w, b.reshape(1, H))


def _mid_body(t_ref, s_ref, w_ref, b_ref, o_ref):
    t = t_ref[0] + t_ref[1]
    s = s_ref[0] + s_ref[1]
    o = t / (s + EPS)
    a = jnp.where(o > 0.0, o, jnp.exp(jnp.minimum(o, 0.0)) - 1.0)
    o_ref[...] = (
        jnp.dot(a, w_ref[...], preferred_element_type=jnp.float32) + b_ref[...]
    )


def _mid(t1, s1, w2d, b2d):
    return pl.pallas_call(
        _mid_body,
        grid=(NB,),
        in_specs=[
            pl.BlockSpec((2, BN, H), lambda i: (0, i, 0)),
            pl.BlockSpec((2, BN, H), lambda i: (0, i, 0)),
            pl.BlockSpec((H, 16), lambda i: (0, 0)),
            pl.BlockSpec((1, 16), lambda i: (0, 0)),
        ],
        out_specs=pl.BlockSpec((BN, 16), lambda i: (i, 0)),
        out_shape=jax.ShapeDtypeStruct((N, 16), jnp.float32),
    )(t1, s1, w2d, b2d)


def _fin_body(ts_ref, o_ref):
    t = ts_ref[0] + ts_ref[1]
    num = t[:, 0:7]
    den = t[:, 8:15]
    o = num / (den + EPS)
    m = jnp.max(o, axis=1, keepdims=True)
    z = o - m
    o_ref[...] = z - jnp.log(jnp.sum(jnp.exp(z), axis=1, keepdims=True))


def _final(ts2):
    return pl.pallas_call(
        _fin_body,
        grid=(NB,),
        in_specs=[pl.BlockSpec((2, BN, 16), lambda i: (0, i, 0))],
        out_specs=pl.BlockSpec((BN, C), lambda i: (i, 0)),
        out_shape=jax.ShapeDtypeStruct((N, C), jnp.float32),
    )(ts2)


def _sc_edge_pass(w_e, h_tab, edge_index, dup):
    W = h_tab.shape[1]
    G = W // 16
    ZR = 624
    TAIL = N - 16 * ZR
    zeros = jnp.zeros((ZR, W), jnp.float32)
    mesh = plsc.VectorSubcoreMesh(core_axis_name="c", subcore_axis_name="s")

    n_acc = 2 if dup else 1
    out_type = [jax.ShapeDtypeStruct((2, N, W), jnp.float32)] * n_acc
    scratch = [
        pltpu.VMEM((2, B), jnp.int32),
        pltpu.VMEM((B, W), jnp.float32),
        pltpu.VMEM((B, W), jnp.float32),
        pltpu.VMEM((B, W), jnp.float32),
        pltpu.VMEM_SHARED((N, W), jnp.float32),
    ]
    if dup:
        scratch.append(pltpu.VMEM_SHARED((N, W), jnp.float32))
    scratch.append(pltpu.SemaphoreType.DMA)

    @functools.partial(
        pl.kernel,
        out_type=out_type,
        mesh=mesh,
        compiler_params=pltpu.CompilerParams(use_tc_tiling_on_sc=False),
        scratch_types=scratch,
    )
    def kern(w_hbm, h_hbm, eidx_hbm, zero_hbm, *refs):
        if dup:
            (t_out, s_out, ebuf, wbuf, rows, stage, t_sh, s_sh, sem) = refs
            accs = ((t_sh, t_out), (s_sh, s_out))
        else:
            (t_out, ebuf, wbuf, rows, stage, t_sh, sem) = refs
            accs = ((t_sh, t_out),)
        cid = lax.axis_index("c")
        sid = lax.axis_index("s")
        wid = sid * 2 + cid
        for sh, _ in accs:
            pltpu.sync_copy(zero_hbm, sh.at[pl.ds(sid * ZR, ZR)])

            @pl.when(sid == 15)
            def _zero_tail():
                pltpu.sync_copy(zero_hbm.at[pl.ds(0, TAIL)],
                                sh.at[pl.ds(16 * ZR, TAIL)])

        plsc.subcore_barrier()
        base = wid * EW

        def chunk_body(i, carry):
            off = base + i * B
            pltpu.sync_copy(eidx_hbm.at[:, pl.ds(off, B)], ebuf)
            pltpu.sync_copy(w_hbm.at[pl.ds(off, B)], wbuf)
            pltpu.async_copy(h_hbm.at[ebuf.at[0]], rows, sem).wait()
            for r in range(B):
                for g in range(G):
                    sl = pl.ds(16 * g, 16)
                    stage[r, sl] = wbuf[r, sl] * rows[r, sl]
            pltpu.sync_copy(stage, t_sh.at[ebuf.at[1]], add=True)
            if dup:
                pltpu.sync_copy(wbuf, s_sh.at[ebuf.at[1]], add=True)
            return carry

        lax.fori_loop(0, EW // B, chunk_body, 0)
        plsc.subcore_barrier()
        for sh, out in accs:
            pltpu.sync_copy(sh.at[pl.ds(sid * ZR, ZR)],
                            out.at[cid, pl.ds(sid * ZR, ZR)])

            @pl.when(sid == 15)
            def _out_tail():
                pltpu.sync_copy(sh.at[pl.ds(16 * ZR, TAIL)],
                                out.at[cid, pl.ds(16 * ZR, TAIL)])

    return kern(w_e, h_tab, edge_index, zeros)


def kernel(x, edge_index, w_mul, lin1_W, lin1_b, mlp1_W1, mlp1_W2, mlp1_b2,
           lin2_W, lin2_b, mlp2_W1, mlp2_W2, mlp2_b2):
    c = w_mul[:, 0]
    c1 = w_mul

    w21d = jnp.zeros((1, 16), jnp.float32)
    w21d = w21d.at[0, 0:7].set(mlp2_W1[0]).at[0, 8:15].set(mlp2_W1[0])
    w22d = jnp.zeros((16, 16), jnp.float32)
    w22d = w22d.at[0:7, 0:7].set(mlp2_W2).at[8:15, 8:15].set(mlp2_W2)
    w2d = jnp.zeros((H, 16), jnp.float32).at[:, 0:7].set(lin2_W)
    b2d = jnp.zeros((1, 16), jnp.float32)
    b2d = b2d.at[0, 0:7].set(lin2_b).at[0, 8:15].set(1.0)

    par1, par2 = _prep(c, mlp1_W1, mlp1_W2, w21d, w22d)
    h1 = _h1_matmul(x, lin1_W, lin1_b)
    w1e = _wexp(c1, par1, H)
    t1, s1 = _sc_edge_pass(w1e, h1, edge_index, dup=True)
    h2 = _mid(t1, s1, w2d, b2d)
    w2e = _wexp(c1, par2, 16)
    ts2 = _sc_edge_pass(w2e, h2, edge_index, dup=False)[0]
    return _final(ts2)


# speedup vs baseline: 1.3875x; 1.3875x over previous
import functools

import jax
import jax.numpy as jnp
from jax import lax
from jax.experimental import pallas as pl
from jax.experimental.pallas import tpu as pltpu
from jax.experimental.pallas import tpu_sc as plsc

N = 10000
E = 320000
F_IN = 128
H = 64
C = 7

NB = 10
BN = N // NB
BE = 2000
NWORK = 32
EW = E // NWORK
B = 40
NCH = EW // B
EPS = 1e-30


def _lrelu_pos(t):
    return jnp.where(t >= 0.0, t, 0.2 * t)


def _lrelu_neg(t):
    return jnp.where(t <= 0.0, t, 0.2 * t)


def _prep_body(c2d, w11, w12, w21d, w22d, par1, par2):
    cabs = jnp.max(jnp.abs(c2d[...]))
    up1 = jnp.dot(_lrelu_pos(w11[...]), w12[...], preferred_element_type=jnp.float32)
    un1 = jnp.dot(_lrelu_neg(w11[...]), w12[...], preferred_element_type=jnp.float32)
    m1 = cabs * jnp.maximum(jnp.abs(up1), jnp.abs(un1))
    par1[...] = jnp.concatenate([up1, un1, m1, jnp.zeros((5, H), jnp.float32)], axis=0)
    up2 = jnp.dot(_lrelu_pos(w21d[...]), w22d[...], preferred_element_type=jnp.float32)
    un2 = jnp.dot(_lrelu_neg(w21d[...]), w22d[...], preferred_element_type=jnp.float32)
    m2 = cabs * jnp.maximum(jnp.abs(up2), jnp.abs(un2))
    par2[...] = jnp.concatenate([up2, un2, m2, jnp.zeros((5, 16), jnp.float32)], axis=0)


def _prep(c, w11, w12, w21d, w22d):
    c2d = c.reshape(2500, 128)
    return pl.pallas_call(
        _prep_body,
        out_shape=(
            jax.ShapeDtypeStruct((8, H), jnp.float32),
            jax.ShapeDtypeStruct((8, 16), jnp.float32),
        ),
    )(c2d, w11, w12, w21d, w22d)


def _wexp_body(c_ref, par_ref, o_ref):
    cb = c_ref[...]
    up = par_ref[0:1, :]
    un = par_ref[1:2, :]
    m = par_ref[2:3, :]
    o_ref[...] = jnp.exp(
        jnp.maximum(cb, 0.0) * up + jnp.minimum(cb, 0.0) * un - m
    )


def _wexp(c1, par, w):
    return pl.pallas_call(
        _wexp_body,
        grid=(E // BE,),
        in_specs=[
            pl.BlockSpec((BE, 1), lambda i: (i, 0)),
            pl.BlockSpec((8, w), lambda i: (0, 0)),
        ],
        out_specs=pl.BlockSpec((BE, w), lambda i: (i, 0)),
        out_shape=jax.ShapeDtypeStruct((E, w), jnp.float32),
    )(c1, par)


def _h1_body(x_ref, w_ref, b_ref, o_ref):
    o_ref[...] = (
        jnp.dot(x_ref[...], w_ref[...], preferred_element_type=jnp.float32)
        + b_ref[...]
    )


def _h1_matmul(x, w, b):
    return pl.pallas_call(
        _h1_body,
        grid=(NB,),
        in_specs=[
            pl.BlockSpec((BN, F_IN), lambda i: (i, 0)),
            pl.BlockSpec((F_IN, H), lambda i: (0, 0)),
            pl.BlockSpec((1, H), lambda i: (0, 0)),
        ],
        out_specs=pl.BlockSpec((BN, H), lambda i: (i, 0)),
        out_shape=jax.ShapeDtypeStruct((N, H), jnp.float32),
    )(x, w, b.reshape(1, H))


def _mid_body(ts_ref, w_ref, b_ref, o_ref):
    t = ts_ref[0] + ts_ref[1]
    num = t[:, :H]
    den = t[:, H:]
    o = num / (den + EPS)
    a = jnp.where(o > 0.0, o, jnp.exp(jnp.minimum(o, 0.0)) - 1.0)
    o_ref[...] = (
        jnp.dot(a, w_ref[...], preferred_element_type=jnp.float32) + b_ref[...]
    )


def _mid(ts1, w2d, b2d):
    return pl.pallas_call(
        _mid_body,
        grid=(NB,),
        in_specs=[
            pl.BlockSpec((2, BN, 2 * H), lambda i: (0, i, 0)),
            pl.BlockSpec((H, 16), lambda i: (0, 0)),
            pl.BlockSpec((1, 16), lambda i: (0, 0)),
        ],
        out_specs=pl.BlockSpec((BN, 16), lambda i: (i, 0)),
        out_shape=jax.ShapeDtypeStruct((N, 16), jnp.float32),
    )(ts1, w2d, b2d)


def _fin_body(ts_ref, o_ref):
    t = ts_ref[0] + ts_ref[1]
    num = t[:, 0:7]
    den = t[:, 8:15]
    o = num / (den + EPS)
    m = jnp.max(o, axis=1, keepdims=True)
    z = o - m
    o_ref[...] = z - jnp.log(jnp.sum(jnp.exp(z), axis=1, keepdims=True))


def _final(ts2):
    return pl.pallas_call(
        _fin_body,
        grid=(NB,),
        in_specs=[pl.BlockSpec((2, BN, 16), lambda i: (0, i, 0))],
        out_specs=pl.BlockSpec((BN, C), lambda i: (i, 0)),
        out_shape=jax.ShapeDtypeStruct((N, C), jnp.float32),
    )(ts2)


def _sc_edge_pass(w_e, h_tab, src3, dst3, dup):
    W = h_tab.shape[1]
    G = W // 16
    SW = 2 * W if dup else W
    ZR = 624
    TAIL = N - 16 * ZR
    zeros = jnp.zeros((ZR, SW), jnp.float32)
    mesh = plsc.VectorSubcoreMesh(core_axis_name="c", subcore_axis_name="s")

    @functools.partial(
        pl.kernel,
        out_type=jax.ShapeDtypeStruct((2, N, SW), jnp.float32),
        mesh=mesh,
        compiler_params=pltpu.CompilerParams(use_tc_tiling_on_sc=False),
        scratch_types=[
            pltpu.VMEM((NCH, B), jnp.int32),
            pltpu.VMEM((NCH, B), jnp.int32),
            pltpu.VMEM((2, B, W), jnp.float32),
            pltpu.VMEM((2, B, W), jnp.float32),
            pltpu.VMEM((2, B, SW), jnp.float32),
            pltpu.VMEM_SHARED((N, SW), jnp.float32),
            pltpu.SemaphoreType.DMA,
            pltpu.SemaphoreType.DMA,
            pltpu.SemaphoreType.DMA,
            pltpu.SemaphoreType.DMA,
            pltpu.SemaphoreType.DMA,
            pltpu.SemaphoreType.DMA,
        ],
    )
    def kern(w_hbm, h_hbm, src_hbm, dst_hbm, zero_hbm, out_hbm,
             sbuf, dbuf, wbuf, rows, stage, ts_sh,
             sw0, sw1, sg0, sg1, st0, st1):
        sws = (sw0, sw1)
        sgs = (sg0, sg1)
        sts = (st0, st1)
        cid = lax.axis_index("c")
        sid = lax.axis_index("s")
        wid = sid * 2 + cid
        pltpu.sync_copy(zero_hbm, ts_sh.at[pl.ds(sid * ZR, ZR)])

        @pl.when(sid == 15)
        def _zero_tail():
            pltpu.sync_copy(zero_hbm.at[pl.ds(0, TAIL)],
                            ts_sh.at[pl.ds(16 * ZR, TAIL)])

        pltpu.sync_copy(src_hbm.at[wid], sbuf)
        pltpu.sync_copy(dst_hbm.at[wid], dbuf)
        plsc.subcore_barrier()

        base = wid * EW

        def start_inputs(b, ch):
            pltpu.async_copy(w_hbm.at[pl.ds(base + ch * B, B)],
                             wbuf.at[b], sws[b])
            pltpu.async_copy(h_hbm.at[sbuf.at[ch]], rows.at[b], sgs[b])

        def wait_inputs(b, ch):
            pltpu.make_async_copy(w_hbm.at[pl.ds(base + ch * B, B)],
                                  wbuf.at[b], sws[b]).wait()
            pltpu.make_async_copy(h_hbm.at[sbuf.at[ch]],
                                  rows.at[b], sgs[b]).wait()

        def drain_scatter(b):
            pltpu.make_async_copy(stage.at[b], ts_sh.at[dbuf.at[0]],
                                  sts[b]).wait()

        start_inputs(0, 0)
        start_inputs(1, 1)

        def super_body(i2, carry):
            for b in range(2):
                ch = 2 * i2 + b

                @pl.when(ch < NCH)
                def _half_step():
                    @pl.when(ch >= 2)
                    def _drain():
                        drain_scatter(b)

                    wait_inputs(b, ch)
                    for r in range(B):
                        for g in range(G):
                            sl = pl.ds(16 * g, 16)
                            wv = wbuf[b, r, sl]
                            stage[b, r, sl] = wv * rows[b, r, sl]
                            if dup:
                                stage[b, r, pl.ds(W + 16 * g, 16)] = wv
                    pltpu.async_copy(stage.at[b], ts_sh.at[dbuf.at[ch]],
                                     sts[b], add=True)

                    @pl.when(ch + 2 < NCH)
                    def _prefetch():
                        start_inputs(b, ch + 2)

            return carry

        lax.fori_loop(0, (NCH + 1) // 2, super_body, 0)
        drain_scatter(0)
        drain_scatter(1)
        plsc.subcore_barrier()
        pltpu.sync_copy(ts_sh.at[pl.ds(sid * ZR, ZR)],
                        out_hbm.at[cid, pl.ds(sid * ZR, ZR)])

        @pl.when(sid == 15)
        def _out_tail():
            pltpu.sync_copy(ts_sh.at[pl.ds(16 * ZR, TAIL)],
                            out_hbm.at[cid, pl.ds(16 * ZR, TAIL)])

    return kern(w_e, h_tab, src3, dst3, zeros)


def kernel(x, edge_index, w_mul, lin1_W, lin1_b, mlp1_W1, mlp1_W2, mlp1_b2,
           lin2_W, lin2_b, mlp2_W1, mlp2_W2, mlp2_b2):
    c = w_mul[:, 0]
    c1 = w_mul
    src3 = edge_index[0].reshape(NWORK, NCH, B)
    dst3 = edge_index[1].reshape(NWORK, NCH, B)

    w21d = jnp.zeros((1, 16), jnp.float32)
    w21d = w21d.at[0, 0:7].set(mlp2_W1[0]).at[0, 8:15].set(mlp2_W1[0])
    w22d = jnp.zeros((16, 16), jnp.float32)
    w22d = w22d.at[0:7, 0:7].set(mlp2_W2).at[8:15, 8:15].set(mlp2_W2)
    w2d = jnp.zeros((H, 16), jnp.float32).at[:, 0:7].set(lin2_W)
    b2d = jnp.zeros((1, 16), jnp.float32)
    b2d = b2d.at[0, 0:7].set(lin2_b).at[0, 8:15].set(1.0)

    par1, par2 = _prep(c, mlp1_W1, mlp1_W2, w21d, w22d)
    h1 = _h1_matmul(x, lin1_W, lin1_b)
    w1e = _wexp(c1, par1, H)
    ts1 = _sc_edge_pass(w1e, h1, src3, dst3, dup=True)
    h2 = _mid(ts1, w2d, b2d)
    w2e = _wexp(c1, par2, 16)
    ts2 = _sc_edge_pass(w2e, h2, src3, dst3, dup=False)
    return _final(ts2)


# speedup vs baseline: 3.6588x; 2.6369x over previous
import functools

import jax
import jax.numpy as jnp
from jax import lax
from jax.experimental import pallas as pl
from jax.experimental.pallas import tpu as pltpu
from jax.experimental.pallas import tpu_sc as plsc

N = 10000
E = 320000
F_IN = 128
H = 64
C = 7

NB = 10
BN = N // NB
NWORK = 32
EW = E // NWORK
B = 40
NCH = EW // B
EPS = 1e-30


def _lrelu_pos(t):
    return jnp.where(t >= 0.0, t, 0.2 * t)


def _lrelu_neg(t):
    return jnp.where(t <= 0.0, t, 0.2 * t)


def _prep_body(c2d, w11, w12, w21d, w22d, par1, par2):
    cabs = jnp.max(jnp.abs(c2d[...]))
    up1 = jnp.dot(_lrelu_pos(w11[...]), w12[...], preferred_element_type=jnp.float32)
    un1 = jnp.dot(_lrelu_neg(w11[...]), w12[...], preferred_element_type=jnp.float32)
    m1 = cabs * jnp.maximum(jnp.abs(up1), jnp.abs(un1))
    par1[...] = jnp.concatenate([up1, un1, m1, jnp.zeros((5, H), jnp.float32)], axis=0)
    up2 = jnp.dot(_lrelu_pos(w21d[...]), w22d[...], preferred_element_type=jnp.float32)
    un2 = jnp.dot(_lrelu_neg(w21d[...]), w22d[...], preferred_element_type=jnp.float32)
    m2 = cabs * jnp.maximum(jnp.abs(up2), jnp.abs(un2))
    par2[...] = jnp.concatenate([up2, un2, m2, jnp.zeros((5, 16), jnp.float32)], axis=0)


def _prep(c2d, w11, w12, w21d, w22d):
    return pl.pallas_call(
        _prep_body,
        out_shape=(
            jax.ShapeDtypeStruct((8, H), jnp.float32),
            jax.ShapeDtypeStruct((8, 16), jnp.float32),
        ),
    )(c2d, w11, w12, w21d, w22d)


def _h1_body(x_ref, w_ref, b_ref, o_ref):
    o_ref[...] = (
        jnp.dot(x_ref[...], w_ref[...], preferred_element_type=jnp.float32)
        + b_ref[...]
    )


def _h1_matmul(x, w, b):
    return pl.pallas_call(
        _h1_body,
        grid=(NB,),
        in_specs=[
            pl.BlockSpec((BN, F_IN), lambda i: (i, 0)),
            pl.BlockSpec((F_IN, H), lambda i: (0, 0)),
            pl.BlockSpec((1, H), lambda i: (0, 0)),
        ],
        out_specs=pl.BlockSpec((BN, H), lambda i: (i, 0)),
        out_shape=jax.ShapeDtypeStruct((N, H), jnp.float32),
    )(x, w, b.reshape(1, H))


def _mid_body(ts_ref, w_ref, b_ref, o_ref):
    t = ts_ref[0] + ts_ref[1]
    num = t[:, :H]
    den = t[:, H:]
    o = num / (den + EPS)
    a = jnp.where(o > 0.0, o, jnp.exp(jnp.minimum(o, 0.0)) - 1.0)
    o_ref[...] = (
        jnp.dot(a, w_ref[...], preferred_element_type=jnp.float32) + b_ref[...]
    )


def _mid(ts1, w2d, b2d):
    return pl.pallas_call(
        _mid_body,
        grid=(NB,),
        in_specs=[
            pl.BlockSpec((2, BN, 2 * H), lambda i: (0, i, 0)),
            pl.BlockSpec((H, 16), lambda i: (0, 0)),
            pl.BlockSpec((1, 16), lambda i: (0, 0)),
        ],
        out_specs=pl.BlockSpec((BN, 16), lambda i: (i, 0)),
        out_shape=jax.ShapeDtypeStruct((N, 16), jnp.float32),
    )(ts1, w2d, b2d)


def _fin_body(ts_ref, o_ref):
    t = ts_ref[0] + ts_ref[1]
    num = t[:, 0:7]
    den = t[:, 8:15]
    o = num / (den + EPS)
    m = jnp.max(o, axis=1, keepdims=True)
    z = o - m
    o_ref[...] = z - jnp.log(jnp.sum(jnp.exp(z), axis=1, keepdims=True))


def _final(ts2):
    return pl.pallas_call(
        _fin_body,
        grid=(NB,),
        in_specs=[pl.BlockSpec((2, BN, 16), lambda i: (0, i, 0))],
        out_specs=pl.BlockSpec((BN, C), lambda i: (i, 0)),
        out_shape=jax.ShapeDtypeStruct((N, C), jnp.float32),
    )(ts2)


def _sc_edge_pass(h_tab, ei4, c2, par, dup):
    W = h_tab.shape[1]
    G = W // 16
    SW = 2 * W if dup else W
    ZR = 624
    TAIL = N - 16 * ZR
    zeros = jnp.zeros((ZR, SW), jnp.float32)
    mesh = plsc.VectorSubcoreMesh(core_axis_name="c", subcore_axis_name="s")

    @functools.partial(
        pl.kernel,
        out_type=jax.ShapeDtypeStruct((2, N, SW), jnp.float32),
        mesh=mesh,
        compiler_params=pltpu.CompilerParams(use_tc_tiling_on_sc=False),
        scratch_types=[
            pltpu.VMEM((NCH, B), jnp.int32),
            pltpu.VMEM((NCH, B), jnp.int32),
            pltpu.VMEM((EW,), jnp.float32),
            pltpu.VMEM((8, W), jnp.float32),
            pltpu.VMEM((2, B, W), jnp.float32),
            pltpu.VMEM((2, B, SW), jnp.float32),
            pltpu.VMEM_SHARED((N, SW), jnp.float32),
            pltpu.SemaphoreType.DMA,
            pltpu.SemaphoreType.DMA,
            pltpu.SemaphoreType.DMA,
            pltpu.SemaphoreType.DMA,
        ],
    )
    def kern(h_hbm, ei_hbm, c_hbm, par_hbm, zero_hbm, out_hbm,
             sbuf, dbuf, cbuf, parv, rows, stage, ts_sh,
             sg0, sg1, st0, st1):
        sgs = (sg0, sg1)
        sts = (st0, st1)
        cid = lax.axis_index("c")
        sid = lax.axis_index("s")
        wid = sid * 2 + cid
        pltpu.sync_copy(zero_hbm, ts_sh.at[pl.ds(sid * ZR, ZR)])

        @pl.when(sid == 15)
        def _zero_tail():
            pltpu.sync_copy(zero_hbm.at[pl.ds(0, TAIL)],
                            ts_sh.at[pl.ds(16 * ZR, TAIL)])

        pltpu.sync_copy(ei_hbm.at[0, wid], sbuf)
        pltpu.sync_copy(ei_hbm.at[1, wid], dbuf)
        pltpu.sync_copy(c_hbm.at[wid], cbuf)
        pltpu.sync_copy(par_hbm, parv)
        plsc.subcore_barrier()

        ups = [parv[0, pl.ds(16 * g, 16)] for g in range(G)]
        uns = [parv[1, pl.ds(16 * g, 16)] for g in range(G)]
        ms = [parv[2, pl.ds(16 * g, 16)] for g in range(G)]

        def start_gather(b, ch):
            pltpu.async_copy(h_hbm.at[sbuf.at[ch]], rows.at[b], sgs[b])

        def wait_gather(b, ch):
            pltpu.make_async_copy(h_hbm.at[sbuf.at[ch]],
                                  rows.at[b], sgs[b]).wait()

        def drain_scatter(b):
            pltpu.make_async_copy(stage.at[b], ts_sh.at[dbuf.at[0]],
                                  sts[b]).wait()

        start_gather(0, 0)
        start_gather(1, 1)

        def super_body(i2, carry):
            for b in range(2):
                ch = 2 * i2 + b

                @pl.when(ch < NCH)
                def _half_step():
                    @pl.when(ch >= 2)
                    def _drain():
                        drain_scatter(b)

                    wait_gather(b, ch)
                    for i3 in range(B // 16):
                        cv = cbuf[pl.ds(ch * B + 16 * i3, 16)]
                        cpv = jnp.maximum(cv, 0.0)
                        cnv = jnp.minimum(cv, 0.0)
                        for j in range(16):
                            r = 16 * i3 + j
                            ap = jnp.full((16,), cpv[j], jnp.float32)
                            an = jnp.full((16,), cnv[j], jnp.float32)
                            for g in range(G):
                                w = jnp.exp(ap * ups[g] + an * uns[g] - ms[g])
                                sl = pl.ds(16 * g, 16)
                                stage[b, r, sl] = w * rows[b, r, sl]
                                if dup:
                                    stage[b, r, pl.ds(W + 16 * g, 16)] = w
                    pltpu.async_copy(stage.at[b], ts_sh.at[dbuf.at[ch]],
                                     sts[b], add=True)

                    @pl.when(ch + 2 < NCH)
                    def _prefetch():
                        start_gather(b, ch + 2)

            return carry

        lax.fori_loop(0, (NCH + 1) // 2, super_body, 0)
        drain_scatter(0)
        drain_scatter(1)
        plsc.subcore_barrier()
        pltpu.sync_copy(ts_sh.at[pl.ds(sid * ZR, ZR)],
                        out_hbm.at[cid, pl.ds(sid * ZR, ZR)])

        @pl.when(sid == 15)
        def _out_tail():
            pltpu.sync_copy(ts_sh.at[pl.ds(16 * ZR, TAIL)],
                            out_hbm.at[cid, pl.ds(16 * ZR, TAIL)])

    return kern(h_tab, ei4, c2, par, zeros)


def kernel(x, edge_index, w_mul, lin1_W, lin1_b, mlp1_W1, mlp1_W2, mlp1_b2,
           lin2_W, lin2_b, mlp2_W1, mlp2_W2, mlp2_b2):
    c2d = w_mul.reshape(2500, 128)
    c2 = w_mul.reshape(NWORK, EW)
    ei4 = edge_index.reshape(2, NWORK, NCH, B)

    w21d = jnp.zeros((1, 16), jnp.float32)
    w21d = w21d.at[0, 0:7].set(mlp2_W1[0]).at[0, 8:15].set(mlp2_W1[0])
    w22d = jnp.zeros((16, 16), jnp.float32)
    w22d = w22d.at[0:7, 0:7].set(mlp2_W2).at[8:15, 8:15].set(mlp2_W2)
    w2d = jnp.zeros((H, 16), jnp.float32).at[:, 0:7].set(lin2_W)
    b2d = jnp.zeros((1, 16), jnp.float32)
    b2d = b2d.at[0, 0:7].set(lin2_b).at[0, 8:15].set(1.0)

    par1, par2 = _prep(c2d, mlp1_W1, mlp1_W2, w21d, w22d)
    h1 = _h1_matmul(x, lin1_W, lin1_b)
    ts1 = _sc_edge_pass(h1, ei4, c2, par1, dup=True)
    h2 = _mid(ts1, w2d, b2d)
    ts2 = _sc_edge_pass(h2, ei4, c2, par2, dup=False)
    return _final(ts2)
